# Initial kernel scaffold; baseline (speedup 1.0000x reference)
import jax
import jax.numpy as jnp
from jax.experimental import pallas as pl


def kernel(mlp_input, mlp_output, W_enc, b_enc, W_dec, b_dec, W_skip, b_skip):
    raise NotImplementedError("write your pallas kernel here")


# speedup vs baseline: 2.1681x; 2.1681x over previous
import functools

import jax
import jax.numpy as jnp
from jax import lax
from jax.experimental import pallas as pl

_K = 32
_NEG = -3.0e38


def _encode_body(x_ref, w_ref, b_ref, out_ref):
    acc = lax.dot_general(x_ref[...], w_ref[...], (((1,), (1,)), ((), ())),
                          preferred_element_type=jnp.float32)
    out_ref[...] = acc + b_ref[...]


def _select_body(pre_ref, hidden_ref, vals_ref, idx_ref, l0_ref, *, rb, h):
    x0 = pre_ref[...]
    col = lax.broadcasted_iota(jnp.int32, (rb, h), 1)
    lane_k = lax.broadcasted_iota(jnp.int32, (rb, _K), 1)

    def body(k, carry):
        x, vals, idxs = carry
        m = jnp.max(x, axis=1, keepdims=True)
        imax = jnp.min(jnp.where(x == m, col, h), axis=1,
                       keepdims=True)
        vals = jnp.where(lane_k == k, m, vals)
        idxs = jnp.where(lane_k == k, imax, idxs)
        x = jnp.where(col == imax, _NEG, x)
        return x, vals, idxs

    init = (x0,
            jnp.zeros((rb, _K), jnp.float32),
            jnp.zeros((rb, _K), jnp.int32))
    xf, vals, idxs = lax.fori_loop(0, _K, body, init)
    hidden = jnp.maximum(pre_ref[...], 0.0) - jnp.maximum(xf, 0.0)
    hidden_ref[...] = hidden
    vals_ref[...] = jnp.maximum(vals, 0.0)
    idx_ref[...] = idxs

    @pl.when(pl.program_id(0) == 0)
    def _():
        l0_ref[...] = jnp.zeros_like(l0_ref)

    l0_ref[...] += jnp.sum((vals > 0.0).astype(jnp.float32), keepdims=True)


def _decode_body(hid_ref, wdec_ref, out_ref):
    @pl.when(pl.program_id(0) == 0)
    def _():
        out_ref[...] = jnp.zeros_like(out_ref)

    out_ref[...] += lax.dot_general(hid_ref[...], wdec_ref[...],
                                    (((1,), (1,)), ((), ())),
                                    preferred_element_type=jnp.float32)


def _skip_body(x_ref, w_ref, b_ref, sp_ref, y_ref, pred_ref, loss_ref):
    pred = lax.dot_general(x_ref[...], w_ref[...], (((1,), (1,)), ((), ())),
                           preferred_element_type=jnp.float32)
    pred = pred + b_ref[...] + sp_ref[...]
    pred_ref[...] = pred
    dif = pred - y_ref[...]

    @pl.when(pl.program_id(0) == 0)
    def _():
        loss_ref[...] = jnp.zeros_like(loss_ref)

    loss_ref[...] += jnp.sum(dif * dif, keepdims=True)


def kernel(mlp_input, mlp_output, W_enc, b_enc, W_dec, b_dec, W_skip, b_skip):
    n, d_in = mlp_input.shape
    h = W_enc.shape[0]
    d_out = W_dec.shape[0]
    f32 = jnp.float32

    hb = 512
    pre = pl.pallas_call(
        _encode_body,
        grid=(h // hb,),
        in_specs=[
            pl.BlockSpec((n, d_in), lambda j: (0, 0)),
            pl.BlockSpec((hb, d_in), lambda j: (j, 0)),
            pl.BlockSpec((1, hb), lambda j: (0, j)),
        ],
        out_specs=pl.BlockSpec((n, hb), lambda j: (0, j)),
        out_shape=jax.ShapeDtypeStruct((n, h), f32),
    )(mlp_input, W_enc, b_enc.reshape(1, h))

    rb = 128
    hidden, vals, idxs, l0_sum = pl.pallas_call(
        functools.partial(_select_body, rb=rb, h=h),
        grid=(n // rb,),
        in_specs=[pl.BlockSpec((rb, h), lambda i: (i, 0))],
        out_specs=[
            pl.BlockSpec((rb, h), lambda i: (i, 0)),
            pl.BlockSpec((rb, _K), lambda i: (i, 0)),
            pl.BlockSpec((rb, _K), lambda i: (i, 0)),
            pl.BlockSpec((1, 1), lambda i: (0, 0)),
        ],
        out_shape=[
            jax.ShapeDtypeStruct((n, h), f32),
            jax.ShapeDtypeStruct((n, _K), f32),
            jax.ShapeDtypeStruct((n, _K), jnp.int32),
            jax.ShapeDtypeStruct((1, 1), f32),
        ],
    )(pre)

    db = 1024
    sparse_out = pl.pallas_call(
        _decode_body,
        grid=(h // db,),
        in_specs=[
            pl.BlockSpec((n, db), lambda j: (0, j)),
            pl.BlockSpec((d_out, db), lambda j: (0, j)),
        ],
        out_specs=pl.BlockSpec((n, d_out), lambda j: (0, 0)),
        out_shape=jax.ShapeDtypeStruct((n, d_out), f32),
    )(hidden, W_dec)

    bias = (b_dec + b_skip).reshape(1, d_out)
    sb = 512
    predicted, loss_sum = pl.pallas_call(
        _skip_body,
        grid=(n // sb,),
        in_specs=[
            pl.BlockSpec((sb, d_in), lambda i: (i, 0)),
            pl.BlockSpec((d_out, d_in), lambda i: (0, 0)),
            pl.BlockSpec((1, d_out), lambda i: (0, 0)),
            pl.BlockSpec((sb, d_out), lambda i: (i, 0)),
            pl.BlockSpec((sb, d_out), lambda i: (i, 0)),
        ],
        out_specs=[
            pl.BlockSpec((sb, d_out), lambda i: (i, 0)),
            pl.BlockSpec((1, 1), lambda i: (0, 0)),
        ],
        out_shape=[
            jax.ShapeDtypeStruct((n, d_out), f32),
            jax.ShapeDtypeStruct((1, 1), f32),
        ],
    )(mlp_input, W_skip, bias, sparse_out, mlp_output)

    loss = (loss_sum[0, 0] / (n * d_out)).astype(f32)
    l0 = (l0_sum[0, 0] / n).astype(f32)
    sparsity_loss = jnp.zeros((), f32)
    del vals, idxs
    return (predicted, hidden, loss, loss, sparsity_loss, l0)
